# baseline (speedup 1.0000x reference)
import functools

import jax
import jax.numpy as jnp
from jax import lax
from jax.experimental import pallas as pl
from jax.experimental.pallas import tpu as pltpu
from jax.experimental.pallas import tpu_sc as plsc

EDIM = 64
NC, NS = 2, 16
NW = NC * NS
CHUNK = 128


def _gather_body(idx_hbm, table_hbm, out_hbm, idx_v, rows_v, sem):
    ch_w = idx_v.shape[0]
    wid = lax.axis_index("s") * NC + lax.axis_index("c")
    cbase = wid * ch_w
    pltpu.sync_copy(idx_hbm.at[wid], idx_v)

    def step(c, carry):
        pltpu.async_copy(table_hbm.at[idx_v.at[c]], rows_v, sem).wait()
        pltpu.sync_copy(rows_v, out_hbm.at[pl.ds((cbase + c) * CHUNK, CHUNK)])
        return carry

    lax.fori_loop(0, ch_w, step, 0)


def _sc_gather(idx_2d, table, n_tot):
    ch_w = idx_2d.shape[1]
    mesh = plsc.VectorSubcoreMesh(core_axis_name="c", subcore_axis_name="s")
    return pl.kernel(
        _gather_body,
        out_type=jax.ShapeDtypeStruct((n_tot, EDIM), jnp.float32),
        mesh=mesh,
        scratch_types=[
            pltpu.VMEM((ch_w, CHUNK), jnp.int32),
            pltpu.VMEM((CHUNK, EDIM), jnp.float32),
            pltpu.SemaphoreType.DMA,
        ],
        compiler_params=pltpu.CompilerParams(use_tc_tiling_on_sc=False),
    )(idx_2d, table)


def _mm_body(x_ref, wt_ref, o_ref):
    o_ref[...] = jnp.dot(x_ref[...], wt_ref[...],
                         preferred_element_type=jnp.float32)


def _tc_project(x, wt):
    n = x.shape[0]
    bm = 4096
    return pl.pallas_call(
        _mm_body,
        grid=(n // bm,),
        in_specs=[
            pl.BlockSpec((bm, EDIM), lambda i: (i, 0)),
            pl.BlockSpec((EDIM, EDIM), lambda i: (0, 0)),
        ],
        out_specs=pl.BlockSpec((bm, EDIM), lambda i: (i, 0)),
        out_shape=jax.ShapeDtypeStruct((n, EDIM), jnp.float32),
    )(x, wt)


def kernel(prem, hypo, table, W):
    B, L = prem.shape
    n_tot = 2 * B * L
    idx = jnp.concatenate([prem.reshape(-1), hypo.reshape(-1)])
    idx_2d = idx.reshape(NW, n_tot // (CHUNK * NW), CHUNK)
    emb = _sc_gather(idx_2d, table, n_tot)
    out = _tc_project(emb, W.T)
    prem_out = out[: B * L].reshape(B, L, EDIM)
    hypo_out = out[B * L:].reshape(B, L, EDIM)
    return (prem_out, hypo_out)


# speedup vs baseline: 1.4046x; 1.4046x over previous
import jax
import jax.numpy as jnp
from jax import lax
from jax.experimental import pallas as pl
from jax.experimental.pallas import tpu as pltpu
from jax.experimental.pallas import tpu_sc as plsc

EDIM = 64
NC, NS = 2, 16
NW = NC * NS
CHUNK = 128
VB = 8192


def _transpose_body(xt_ref, o_ref):
    o_ref[...] = xt_ref[...].T


def _tc_table_rowmajor(table_t):
    v = table_t.shape[1]
    return pl.pallas_call(
        _transpose_body,
        grid=(pl.cdiv(v, VB),),
        in_specs=[pl.BlockSpec((EDIM, VB), lambda i: (0, i))],
        out_specs=pl.BlockSpec((VB, EDIM), lambda i: (i, 0)),
        out_shape=jax.ShapeDtypeStruct((v, EDIM), jnp.float32),
    )(table_t)


def _gather_body(idx_hbm, table_hbm, out_hbm, idx_v, rows_v, sem):
    ch_w = idx_v.shape[0]
    wid = lax.axis_index("s") * NC + lax.axis_index("c")
    cbase = wid * ch_w
    pltpu.sync_copy(idx_hbm.at[wid], idx_v)

    def step(c, carry):
        pltpu.async_copy(table_hbm.at[idx_v.at[c]], rows_v, sem).wait()
        pltpu.sync_copy(rows_v, out_hbm.at[pl.ds((cbase + c) * CHUNK, CHUNK)])
        return carry

    lax.fori_loop(0, ch_w, step, 0)


def _sc_gather(idx_3d, table_rm):
    n = idx_3d.shape[0] * idx_3d.shape[1] * idx_3d.shape[2]
    ch_w = idx_3d.shape[1]
    mesh = plsc.VectorSubcoreMesh(core_axis_name="c", subcore_axis_name="s")
    return pl.kernel(
        _gather_body,
        out_type=jax.ShapeDtypeStruct((n, EDIM), jnp.float32),
        mesh=mesh,
        scratch_types=[
            pltpu.VMEM((ch_w, CHUNK), jnp.int32),
            pltpu.VMEM((CHUNK, EDIM), jnp.float32),
            pltpu.SemaphoreType.DMA,
        ],
        compiler_params=pltpu.CompilerParams(use_tc_tiling_on_sc=False),
    )(idx_3d, table_rm)


def _mm_body(x_ref, w_ref, o_ref):
    o_ref[0] = jax.lax.dot_general(
        w_ref[...], x_ref[0], (((1,), (1,)), ((), ())),
        preferred_element_type=jnp.float32)


def _tc_project_t(emb_3d, w):
    l, b, _ = emb_3d.shape
    return pl.pallas_call(
        _mm_body,
        grid=(l,),
        in_specs=[
            pl.BlockSpec((1, b, EDIM), lambda i: (i, 0, 0)),
            pl.BlockSpec((EDIM, EDIM), lambda i: (0, 0)),
        ],
        out_specs=pl.BlockSpec((1, EDIM, b), lambda i: (i, 0, 0)),
        out_shape=jax.ShapeDtypeStruct((l, EDIM, b), jnp.float32),
    )(emb_3d, w)


def kernel(prem, hypo, table, W):
    B, L = prem.shape
    n = B * L
    table_rm = _tc_table_rowmajor(table.T)

    outs = []
    for ind in (prem, hypo):
        idx_3d = ind.T.reshape(NW, n // (NW * CHUNK), CHUNK)
        emb = _sc_gather(idx_3d, table_rm)
        out_t = _tc_project_t(emb.reshape(L, B, EDIM), W)
        outs.append(out_t.transpose(2, 0, 1))
    return (outs[0], outs[1])


# speedup vs baseline: 1.6977x; 1.2086x over previous
import jax
import jax.numpy as jnp
from jax import lax
from jax.experimental import pallas as pl
from jax.experimental.pallas import tpu as pltpu
from jax.experimental.pallas import tpu_sc as plsc

EDIM = 64
NC, NS = 2, 16
NW = NC * NS
CHUNK = 128


def _gather_body(idx_hbm, table_hbm, out_hbm, idx_v, rows_v, sem):
    ch_w = idx_v.shape[0]
    wid = lax.axis_index("s") * NC + lax.axis_index("c")
    cbase = wid * ch_w
    pltpu.sync_copy(idx_hbm.at[wid], idx_v)

    def step(c, carry):
        pltpu.async_copy(table_hbm.at[idx_v.at[c]], rows_v, sem).wait()
        pltpu.sync_copy(rows_v, out_hbm.at[pl.ds((cbase + c) * CHUNK, CHUNK)])
        return carry

    lax.fori_loop(0, ch_w, step, 0)


def _sc_gather(idx_3d, table):
    n = idx_3d.shape[0] * idx_3d.shape[1] * idx_3d.shape[2]
    ch_w = idx_3d.shape[1]
    mesh = plsc.VectorSubcoreMesh(core_axis_name="c", subcore_axis_name="s")
    return pl.kernel(
        _gather_body,
        out_type=jax.ShapeDtypeStruct((n, EDIM), jnp.float32),
        mesh=mesh,
        scratch_types=[
            pltpu.VMEM((ch_w, CHUNK), jnp.int32),
            pltpu.VMEM((CHUNK, EDIM), jnp.float32),
            pltpu.SemaphoreType.DMA,
        ],
        compiler_params=pltpu.CompilerParams(use_tc_tiling_on_sc=False),
    )(idx_3d, table)


def _mm_body(x_ref, w_ref, o_ref):
    hb = x_ref.shape[1]
    w = w_ref[...]
    x = x_ref[0]
    dn = (((1,), (1,)), ((), ()))
    o_ref[0, :, :hb] = jax.lax.dot_general(
        w, x[:, :EDIM], dn, preferred_element_type=jnp.float32)
    o_ref[0, :, hb:] = jax.lax.dot_general(
        w, x[:, EDIM:], dn, preferred_element_type=jnp.float32)


def _tc_project_t(emb, w, l, b):
    x128 = emb.reshape(l, b // 2, 2 * EDIM)
    return pl.pallas_call(
        _mm_body,
        grid=(l,),
        in_specs=[
            pl.BlockSpec((1, b // 2, 2 * EDIM), lambda i: (i, 0, 0)),
            pl.BlockSpec((EDIM, EDIM), lambda i: (0, 0)),
        ],
        out_specs=pl.BlockSpec((1, EDIM, b), lambda i: (i, 0, 0)),
        out_shape=jax.ShapeDtypeStruct((l, EDIM, b), jnp.float32),
    )(x128, w)


def kernel(prem, hypo, table, W):
    B, L = prem.shape
    n = B * L
    outs = []
    for ind in (prem, hypo):
        idx_pairs = ind.T.reshape(L, 2, B // 2).transpose(0, 2, 1)
        idx_3d = idx_pairs.reshape(NW, n // (NW * CHUNK), CHUNK)
        emb = _sc_gather(idx_3d, table)
        out_t = _tc_project_t(emb, W, L, B)
        outs.append(out_t.transpose(2, 0, 1))
    return (outs[0], outs[1])
